# baseline (speedup 1.0000x reference)
import functools

import jax
import jax.numpy as jnp
from jax import lax
from jax.experimental import pallas as pl
from jax.experimental.pallas import tpu as pltpu
from jax.experimental.pallas import tpu_sc as plsc

_TN = 6144
_KP = 72
_LANES = 128


def _sc_gather_aug(embT, x):
    D, V = embT.shape
    B = x.shape[0]
    info = plsc.get_sparse_core_info()
    nw = info.num_cores * info.num_subcores
    b_per_w = B // nw

    mesh = plsc.VectorSubcoreMesh(core_axis_name="c", subcore_axis_name="s")

    @functools.partial(
        pl.kernel,
        mesh=mesh,
        compiler_params=pltpu.CompilerParams(needs_layout_passes=False),
        out_type=jax.ShapeDtypeStruct((B, _KP), jnp.float32),
        scratch_types=[
            pltpu.VMEM((b_per_w,), jnp.int32),
            pltpu.VMEM((D, _LANES), jnp.float32),
            pltpu.VMEM((D, _LANES), jnp.float32),
            pltpu.VMEM((D, _LANES), jnp.float32),
            pltpu.VMEM((D, _LANES), jnp.float32),
            pltpu.VMEM((b_per_w, _KP), jnp.float32),
            pltpu.SemaphoreType.DMA,
            pltpu.SemaphoreType.DMA,
            pltpu.SemaphoreType.DMA,
            pltpu.SemaphoreType.DMA,
        ],
    )
    def gather_kernel(
        table_hbm, idx_hbm, out_hbm, idx_v, buf0, buf1, buf2, buf3, rows_v,
        sem0, sem1, sem2, sem3,
    ):
        wid = lax.axis_index("s") * info.num_cores + lax.axis_index("c")
        base = wid * b_per_w
        pltpu.sync_copy(idx_hbm.at[pl.ds(base, b_per_w)], idx_v)
        bufs = (buf0, buf1, buf2, buf3)
        sems = (sem0, sem1, sem2, sem3)
        nbuf = len(bufs)
        iota16 = lax.iota(jnp.int32, 16)
        ones16 = jnp.ones((16,), jnp.float32)
        zeros16 = jnp.zeros((16,), jnp.float32)

        for col in range(D, _KP):
            fill = ones16 if col == D else zeros16
            for t in range(b_per_w // 16):
                plsc.store_scatter(
                    rows_v,
                    [iota16 + 16 * t, jnp.full((16,), col, jnp.int32)],
                    fill,
                )

        idx_vecs = [idx_v[pl.ds(g * 16, 16)] for g in range(b_per_w // 16)]

        def start(j):
            r = idx_vecs[j // 16][j % 16]
            c = pl.multiple_of((r // _LANES) * _LANES, _LANES)
            pltpu.async_copy(
                table_hbm.at[:, pl.ds(c, _LANES)], bufs[j % nbuf], sems[j % nbuf]
            )

        def extract(j):
            r = idx_vecs[j // 16][j % 16]
            lane = jnp.full((16,), lax.rem(r, _LANES), jnp.int32)
            pltpu.make_async_copy(
                table_hbm.at[:, pl.ds(0, _LANES)], bufs[j % nbuf], sems[j % nbuf]
            ).wait()
            rowj = jnp.full((16,), j, jnp.int32)
            for k4 in range(D // 16):
                cols = iota16 + (16 * k4)
                vals = plsc.load_gather(bufs[j % nbuf], [cols, lane])
                plsc.store_scatter(rows_v, [rowj, cols], vals)

        for j in range(nbuf - 1):
            start(j)
        for j in range(b_per_w):
            if j + nbuf - 1 < b_per_w:
                start(j + nbuf - 1)
            extract(j)

        pltpu.sync_copy(rows_v, out_hbm.at[pl.ds(base, b_per_w)])

    return gather_kernel(embT, x)


def _tc_project_t(e_aug, Wt, b_row):
    B = e_aug.shape[0]
    D, V = Wt.shape

    def body(w_ref, b_ref, e_ref, o_ref):
        w_aug = jnp.concatenate(
            [w_ref[...], b_ref[...], jnp.zeros((_KP - D - 1, _TN), jnp.float32)],
            axis=0,
        )
        o_ref[...] = lax.dot_general(
            w_aug,
            e_ref[...],
            (((0,), (1,)), ((), ())),
            preferred_element_type=jnp.float32,
        )

    return pl.pallas_call(
        body,
        grid=(pl.cdiv(V, _TN),),
        in_specs=[
            pl.BlockSpec((D, _TN), lambda i: (0, i)),
            pl.BlockSpec((1, _TN), lambda i: (0, i)),
            pl.BlockSpec((B, _KP), lambda i: (0, 0)),
        ],
        out_specs=pl.BlockSpec((_TN, B), lambda i: (i, 0)),
        out_shape=jax.ShapeDtypeStruct((V, B), jnp.float32),
    )(Wt, b_row, e_aug)


def kernel(x, emb, W, b):
    e_aug = _sc_gather_aug(emb.T, x)
    out_t = _tc_project_t(e_aug, W.T, b.reshape(1, -1))
    return out_t.T


# speedup vs baseline: 1.0121x; 1.0121x over previous
import functools

import jax
import jax.numpy as jnp
from jax import lax
from jax.experimental import pallas as pl
from jax.experimental.pallas import tpu as pltpu
from jax.experimental.pallas import tpu_sc as plsc

_TN = 4096
_KP = 72
_LANES = 128


def _sc_gather_aug(embT, x):
    D, V = embT.shape
    B = x.shape[0]
    info = plsc.get_sparse_core_info()
    nw = info.num_cores * info.num_subcores
    b_per_w = B // nw

    mesh = plsc.VectorSubcoreMesh(core_axis_name="c", subcore_axis_name="s")

    @functools.partial(
        pl.kernel,
        mesh=mesh,
        compiler_params=pltpu.CompilerParams(needs_layout_passes=False),
        out_type=jax.ShapeDtypeStruct((B, _KP), jnp.float32),
        scratch_types=[
            pltpu.VMEM((b_per_w,), jnp.int32),
            *[pltpu.VMEM((D, _LANES), jnp.float32) for _ in range(8)],
            pltpu.VMEM((b_per_w, _KP), jnp.float32),
            *[pltpu.SemaphoreType.DMA for _ in range(8)],
        ],
    )
    def gather_kernel(table_hbm, idx_hbm, out_hbm, idx_v, *rest):
        bufs = rest[:8]
        rows_v = rest[8]
        sems = rest[9:17]
        nbuf = len(bufs)
        wid = lax.axis_index("s") * info.num_cores + lax.axis_index("c")
        base = wid * b_per_w
        pltpu.sync_copy(idx_hbm.at[pl.ds(base, b_per_w)], idx_v)
        iota16 = lax.iota(jnp.int32, 16)
        ones16 = jnp.ones((16,), jnp.float32)
        zeros16 = jnp.zeros((16,), jnp.float32)

        for col in range(D, _KP):
            fill = ones16 if col == D else zeros16
            for t in range(b_per_w // 16):
                plsc.store_scatter(
                    rows_v,
                    [iota16 + 16 * t, jnp.full((16,), col, jnp.int32)],
                    fill,
                )

        idx_vecs = [idx_v[pl.ds(g * 16, 16)] for g in range(b_per_w // 16)]

        def start(j):
            r = idx_vecs[j // 16][j % 16]
            c = pl.multiple_of((r // _LANES) * _LANES, _LANES)
            pltpu.async_copy(
                table_hbm.at[:, pl.ds(c, _LANES)], bufs[j % nbuf], sems[j % nbuf]
            )

        def extract(j):
            r = idx_vecs[j // 16][j % 16]
            lane = jnp.full((16,), lax.rem(r, _LANES), jnp.int32)
            pltpu.make_async_copy(
                table_hbm.at[:, pl.ds(0, _LANES)], bufs[j % nbuf], sems[j % nbuf]
            ).wait()
            rowj = jnp.full((16,), j, jnp.int32)
            for k4 in range(D // 16):
                cols = iota16 + (16 * k4)
                vals = plsc.load_gather(bufs[j % nbuf], [cols, lane])
                plsc.store_scatter(rows_v, [rowj, cols], vals)

        for j in range(nbuf - 1):
            start(j)
        for j in range(b_per_w):
            if j + nbuf - 1 < b_per_w:
                start(j + nbuf - 1)
            extract(j)

        pltpu.sync_copy(rows_v, out_hbm.at[pl.ds(base, b_per_w)])

    return gather_kernel(embT, x)


def _tc_project_t(e_aug, Wt, b_row):
    B = e_aug.shape[0]
    D, V = Wt.shape

    def body(w_ref, b_ref, e_ref, o_ref):
        w_aug = jnp.concatenate(
            [w_ref[...], b_ref[...], jnp.zeros((_KP - D - 1, _TN), jnp.float32)],
            axis=0,
        )
        o_ref[...] = lax.dot_general(
            w_aug,
            e_ref[...],
            (((0,), (1,)), ((), ())),
            preferred_element_type=jnp.float32,
        )

    return pl.pallas_call(
        body,
        grid=(pl.cdiv(V, _TN),),
        in_specs=[
            pl.BlockSpec((D, _TN), lambda i: (0, i)),
            pl.BlockSpec((1, _TN), lambda i: (0, i)),
            pl.BlockSpec((B, _KP), lambda i: (0, 0)),
        ],
        out_specs=pl.BlockSpec((_TN, B), lambda i: (i, 0)),
        out_shape=jax.ShapeDtypeStruct((V, B), jnp.float32),
    )(Wt, b_row, e_aug)


def kernel(x, emb, W, b):
    e_aug = _sc_gather_aug(emb.T, x)
    out_t = _tc_project_t(e_aug, W.T, b.reshape(1, -1))
    return out_t.T
